# baseline (speedup 1.0000x reference)
import jax
import jax.numpy as jnp
import numpy as np
from jax.experimental import pallas as pl

N = 2048
N_RBF = 16
N_HIDDEN = 64
CUTOFF = 5.0
R_MIN = 0.5
BLK = 256

_CENTERS = np.linspace(R_MIN, CUTOFF, N_RBF).astype(np.float32)
_ETA = np.float32(0.5 * (CUTOFF - R_MIN) / N_RBF)
_INV2ETA2 = np.float32(1.0 / (2.0 * _ETA * _ETA))
_PI = np.float32(np.pi)


def _fused_kernel(pos_blk_ref, pos_t_ref, w1_ref, b1_ref, w2_ref, b2_ref,
                  w3_ref, b3_ref, out_ref):
    i = pl.program_id(0)
    xi = pos_blk_ref[:, 0:1]
    yi = pos_blk_ref[:, 1:2]
    zi = pos_blk_ref[:, 2:3]
    xj = pos_t_ref[0:1, :]
    yj = pos_t_ref[1:2, :]
    zj = pos_t_ref[2:3, :]
    dx = xj - xi
    dy = yj - yi
    dz = zj - zi
    sq = dx * dx + dy * dy + dz * dz
    pos_mask = sq > 0.0
    dist = jnp.where(pos_mask, jnp.sqrt(jnp.where(pos_mask, sq, 1.0)), 0.0)
    x = dist * np.float32(1.0 / CUTOFF)
    smooth = jnp.where(x < 1.0, 0.5 * (1.0 + jnp.cos(_PI * x)), 0.0)
    w = jnp.where(pos_mask & (dist < CUTOFF), smooth, 0.0)
    cols = []
    for k in range(N_RBF):
        t = dist - _CENTERS[k]
        r = jnp.exp(-(t * t) * _INV2ETA2) * w
        cols.append(jnp.sum(r, axis=1, keepdims=True))
    features = jnp.concatenate(cols, axis=1)
    h = features @ w1_ref[...] + b1_ref[...]
    h = jax.nn.silu(h)
    h = h @ w2_ref[...] + b2_ref[...]
    h = jax.nn.silu(h)
    atomic_e = h @ w3_ref[...] + b3_ref[...]
    partial = jnp.sum(atomic_e).reshape(1, 1)

    @pl.when(i == 0)
    def _():
        out_ref[...] = jnp.zeros((1, 1), jnp.float32)

    out_ref[...] += partial


def kernel(positions, W1, b1, W2, b2, W3, b3):
    pos_t = positions.T
    b1r = b1.reshape(1, N_HIDDEN)
    b2r = b2.reshape(1, N_HIDDEN)
    b3r = b3.reshape(1, 1)
    out = pl.pallas_call(
        _fused_kernel,
        grid=(N // BLK,),
        in_specs=[
            pl.BlockSpec((BLK, 3), lambda i: (i, 0)),
            pl.BlockSpec((3, N), lambda i: (0, 0)),
            pl.BlockSpec((N_RBF, N_HIDDEN), lambda i: (0, 0)),
            pl.BlockSpec((1, N_HIDDEN), lambda i: (0, 0)),
            pl.BlockSpec((N_HIDDEN, N_HIDDEN), lambda i: (0, 0)),
            pl.BlockSpec((1, N_HIDDEN), lambda i: (0, 0)),
            pl.BlockSpec((N_HIDDEN, 1), lambda i: (0, 0)),
            pl.BlockSpec((1, 1), lambda i: (0, 0)),
        ],
        out_specs=pl.BlockSpec((1, 1), lambda i: (0, 0)),
        out_shape=jax.ShapeDtypeStruct((1, 1), jnp.float32),
    )(positions, pos_t, W1, b1r, W2, b2r, W3, b3r)
    return out[0, 0]


# speedup vs baseline: 1.5298x; 1.5298x over previous
import jax
import jax.numpy as jnp
import numpy as np
from jax.experimental import pallas as pl

N = 2048
N_RBF = 16
N_HIDDEN = 64
CUTOFF = 5.0
R_MIN = 0.5
BLK = 256

_CENTERS = np.linspace(R_MIN, CUTOFF, N_RBF).astype(np.float32)
_ETA = np.float32(0.5 * (CUTOFF - R_MIN) / N_RBF)
_INV2ETA2 = np.float32(1.0 / (2.0 * _ETA * _ETA))
_PI = np.float32(np.pi)


def _fused_kernel(pos_blk_ref, pos_t_ref, w1_ref, b1_ref, w2_ref, b2_ref,
                  w3_ref, b3_ref, out_ref):
    i = pl.program_id(0)
    xi = pos_blk_ref[:, 0:1]
    yi = pos_blk_ref[:, 1:2]
    zi = pos_blk_ref[:, 2:3]
    xj = pos_t_ref[0:1, :]
    yj = pos_t_ref[1:2, :]
    zj = pos_t_ref[2:3, :]
    dx = xj - xi
    dy = yj - yi
    dz = zj - zi
    sq = dx * dx + dy * dy + dz * dz
    pos_mask = sq > 0.0
    dist = jnp.where(pos_mask, jnp.sqrt(jnp.where(pos_mask, sq, 1.0)), 0.0)
    x = dist * np.float32(1.0 / CUTOFF)
    smooth = jnp.where(x < 1.0, 0.5 * (1.0 + jnp.cos(_PI * x)), 0.0)
    w = jnp.where(pos_mask & (dist < CUTOFF), smooth, 0.0)
    logw = jnp.log(w)
    ds = dist * np.float32(np.sqrt(_INV2ETA2))
    ones_col = jnp.ones((N, 1), jnp.float32)
    cols = []
    for k in range(N_RBF):
        t = ds - np.float32(_CENTERS[k] * np.sqrt(_INV2ETA2))
        r = jnp.exp(logw - t * t)
        cols.append(jax.lax.dot(r, ones_col,
                                preferred_element_type=jnp.float32))
    features = jnp.concatenate(cols, axis=1)
    h = features @ w1_ref[...] + b1_ref[...]
    h = jax.nn.silu(h)
    h = h @ w2_ref[...] + b2_ref[...]
    h = jax.nn.silu(h)
    atomic_e = h @ w3_ref[...] + b3_ref[...]
    partial = jnp.sum(atomic_e).reshape(1, 1)

    @pl.when(i == 0)
    def _():
        out_ref[...] = jnp.zeros((1, 1), jnp.float32)

    out_ref[...] += partial


def kernel(positions, W1, b1, W2, b2, W3, b3):
    pos_t = positions.T
    b1r = b1.reshape(1, N_HIDDEN)
    b2r = b2.reshape(1, N_HIDDEN)
    b3r = b3.reshape(1, 1)
    out = pl.pallas_call(
        _fused_kernel,
        grid=(N // BLK,),
        in_specs=[
            pl.BlockSpec((BLK, 3), lambda i: (i, 0)),
            pl.BlockSpec((3, N), lambda i: (0, 0)),
            pl.BlockSpec((N_RBF, N_HIDDEN), lambda i: (0, 0)),
            pl.BlockSpec((1, N_HIDDEN), lambda i: (0, 0)),
            pl.BlockSpec((N_HIDDEN, N_HIDDEN), lambda i: (0, 0)),
            pl.BlockSpec((1, N_HIDDEN), lambda i: (0, 0)),
            pl.BlockSpec((N_HIDDEN, 1), lambda i: (0, 0)),
            pl.BlockSpec((1, 1), lambda i: (0, 0)),
        ],
        out_specs=pl.BlockSpec((1, 1), lambda i: (0, 0)),
        out_shape=jax.ShapeDtypeStruct((1, 1), jnp.float32),
    )(positions, pos_t, W1, b1r, W2, b2r, W3, b3r)
    return out[0, 0]


# speedup vs baseline: 2.5324x; 1.6554x over previous
import jax
import jax.numpy as jnp
import numpy as np
from jax.experimental import pallas as pl

N = 2048
N_RBF = 16
N_HIDDEN = 64
CUTOFF = 5.0
R_MIN = 0.5
BLK = 256

_CENTERS = np.linspace(R_MIN, CUTOFF, N_RBF).astype(np.float32)
_ETA = np.float32(0.5 * (CUTOFF - R_MIN) / N_RBF)
_INV2ETA2 = np.float32(1.0 / (2.0 * _ETA * _ETA))
_PI = np.float32(np.pi)
_SINC0 = np.float32(-0.5 * np.pi)
_SINC1 = np.float32(0.5 * np.pi ** 3 / 6.0)
_SINC2 = np.float32(-0.5 * np.pi ** 5 / 120.0)
_SINC3 = np.float32(0.5 * np.pi ** 7 / 5040.0)
_SINC4 = np.float32(-0.5 * np.pi ** 9 / 362880.0)
_SINC5 = np.float32(0.5 * np.pi ** 11 / 39916800.0)


def _fused_kernel(pos_blk_ref, pos_t_ref, w1_ref, b1_ref, w2_ref, b2_ref,
                  w3_ref, b3_ref, out_ref):
    i = pl.program_id(0)
    xi = pos_blk_ref[:, 0:1]
    yi = pos_blk_ref[:, 1:2]
    zi = pos_blk_ref[:, 2:3]
    xj = pos_t_ref[0:1, :]
    yj = pos_t_ref[1:2, :]
    zj = pos_t_ref[2:3, :]
    dx = xj - xi
    dy = yj - yi
    dz = zj - zi
    sq = dx * dx + dy * dy + dz * dz
    dist = jnp.sqrt(sq)
    x = dist * np.float32(1.0 / CUTOFF)
    z = x - np.float32(0.5)
    s = z * z
    q = _SINC5
    q = q * s + _SINC4
    q = q * s + _SINC3
    q = q * s + _SINC2
    q = q * s + _SINC1
    q = q * s + _SINC0
    smooth = jnp.maximum(np.float32(0.5) + z * q, 0.0)
    w = jnp.where((sq > 0.0) & (dist < CUTOFF), smooth, 0.0)
    logw = jnp.log(w)
    ds = dist * np.float32(np.sqrt(_INV2ETA2))
    ones_col = jnp.ones((N, 1), jnp.float32)
    cols = []
    for k in range(N_RBF):
        t = ds - np.float32(_CENTERS[k] * np.sqrt(_INV2ETA2))
        r = jnp.exp(logw - t * t)
        cols.append(jax.lax.dot(r, ones_col,
                                preferred_element_type=jnp.float32))
    features = jnp.concatenate(cols, axis=1)
    h = features @ w1_ref[...] + b1_ref[...]
    h = jax.nn.silu(h)
    h = h @ w2_ref[...] + b2_ref[...]
    h = jax.nn.silu(h)
    atomic_e = h @ w3_ref[...] + b3_ref[...]
    partial = jnp.sum(atomic_e).reshape(1, 1)

    @pl.when(i == 0)
    def _():
        out_ref[...] = jnp.zeros((1, 1), jnp.float32)

    out_ref[...] += partial


def kernel(positions, W1, b1, W2, b2, W3, b3):
    pos_t = positions.T
    b1r = b1.reshape(1, N_HIDDEN)
    b2r = b2.reshape(1, N_HIDDEN)
    b3r = b3.reshape(1, 1)
    out = pl.pallas_call(
        _fused_kernel,
        grid=(N // BLK,),
        in_specs=[
            pl.BlockSpec((BLK, 3), lambda i: (i, 0)),
            pl.BlockSpec((3, N), lambda i: (0, 0)),
            pl.BlockSpec((N_RBF, N_HIDDEN), lambda i: (0, 0)),
            pl.BlockSpec((1, N_HIDDEN), lambda i: (0, 0)),
            pl.BlockSpec((N_HIDDEN, N_HIDDEN), lambda i: (0, 0)),
            pl.BlockSpec((1, N_HIDDEN), lambda i: (0, 0)),
            pl.BlockSpec((N_HIDDEN, 1), lambda i: (0, 0)),
            pl.BlockSpec((1, 1), lambda i: (0, 0)),
        ],
        out_specs=pl.BlockSpec((1, 1), lambda i: (0, 0)),
        out_shape=jax.ShapeDtypeStruct((1, 1), jnp.float32),
    )(positions, pos_t, W1, b1r, W2, b2r, W3, b3r)
    return out[0, 0]
